# baseline (speedup 1.0000x reference)
import functools

import jax
import jax.numpy as jnp
from jax import lax
from jax.experimental import pallas as pl
from jax.experimental.pallas import tpu as pltpu
from jax.experimental.pallas import tpu_sc as plsc

_BB = 2048
_CP = 1008
_SC_ROWS = 2048
_CHUNK = 32


def _colmax_body(x_ref, o_ref):
    i = pl.program_id(0)
    x = x_ref[...]
    rowmax = jnp.max(x, axis=1, keepdims=True)
    cm = jnp.max(x - rowmax, axis=0, keepdims=True)

    @pl.when(i == 0)
    def _():
        o_ref[...] = cm

    @pl.when(i != 0)
    def _():
        o_ref[...] = jnp.maximum(o_ref[...], cm)


def _loss_body(cm_ref, cnt_ref, scm_ref, acc_ref, o_ref):
    c = cm_ref.shape[0]
    cnt = jnp.sum(cnt_ref[...], axis=0, keepdims=True)[:, :c]
    scm = jnp.sum(scm_ref[...], axis=0, keepdims=True)[:, :c]
    w = lax.dot_general(
        cnt,
        cm_ref[...],
        (((1,), (0,)), ((), ())),
        precision=lax.Precision.HIGHEST,
        preferred_element_type=jnp.float32,
    )
    mask = jnp.maximum(
        (acc_ref[...] == 0.0).astype(jnp.float32),
        (scm > 0.0).astype(jnp.float32),
    )
    o_ref[...] = -jnp.sum(w * mask, keepdims=True)


@functools.lru_cache(maxsize=None)
def _make_sc_kernel(nw, nc, nb, c, sc_rows, row0):
    mesh = plsc.VectorSubcoreMesh(core_axis_name="c", subcore_axis_name="s")
    nr = sc_rows // nw
    n_chunks = nr // _CHUNK
    unroll = 4
    assert c % unroll == 0

    @functools.partial(
        pl.kernel,
        mesh=mesh,
        out_type=(
            jax.ShapeDtypeStruct((nw, _CP), jnp.float32),
            jax.ShapeDtypeStruct((nw, _CP), jnp.float32),
        ),
        scratch_types=[
            pltpu.VMEM((nb,), jnp.int32),
            pltpu.VMEM((_CP,), jnp.float32),
            pltpu.VMEM((_CP,), jnp.float32),
            pltpu.VMEM((_CHUNK, c), jnp.float32),
        ],
        compiler_params=pltpu.CompilerParams(needs_layout_passes=False),
    )
    def sc_k(labels_hbm, out_hbm, cnt_out, mask_out, idx_v, hist_v, mhist_v,
             buf_v):
        wid = lax.axis_index("s") * nc + lax.axis_index("c")
        zeros = jnp.zeros((16,), jnp.float32)
        ones = jnp.ones((16,), jnp.float32)
        for j in range(_CP // 16):
            hist_v[pl.ds(j * 16, 16)] = zeros
            mhist_v[pl.ds(j * 16, 16)] = zeros

        pltpu.sync_copy(labels_hbm.at[pl.ds(wid * nb, nb)], idx_v)
        for i in range(nb // 16):
            plsc.addupdate_scatter(hist_v, [idx_v[pl.ds(i * 16, 16)]], ones)
        pltpu.sync_copy(hist_v, cnt_out.at[wid])

        lanes = lax.iota(jnp.int32, 16)
        neg_inf = jnp.full((16,), -jnp.inf, jnp.float32)
        izeros = jnp.zeros((16,), jnp.int32)
        for ch in range(n_chunks):
            r0 = row0 + wid * nr + ch * _CHUNK
            pltpu.sync_copy(out_hbm.at[pl.ds(r0, _CHUNK)], buf_v)
            for g in range(_CHUNK // 16):
                rows = lanes + g * 16

                def body(ci, carry):
                    maxs, idxs = carry
                    new_m, new_i = [], []
                    for k in range(unroll):
                        cv = jnp.full((16,), ci * unroll + k, jnp.int32)
                        x = plsc.load_gather(buf_v, [rows, cv])
                        gt = x > maxs[k]
                        new_m.append(jnp.where(gt, x, maxs[k]))
                        new_i.append(jnp.where(gt, cv, idxs[k]))
                    return tuple(new_m), tuple(new_i)

                maxs, idxs = lax.fori_loop(
                    0, c // unroll, body,
                    ((neg_inf,) * unroll, (izeros,) * unroll))
                m, ix = maxs[0], idxs[0]
                for k in range(1, unroll):
                    take = (maxs[k] > m) | ((maxs[k] == m) & (idxs[k] < ix))
                    m = jnp.where(take, maxs[k], m)
                    ix = jnp.where(take, idxs[k], ix)
                plsc.addupdate_scatter(mhist_v, [ix], ones)
        pltpu.sync_copy(mhist_v, mask_out.at[wid])

    return sc_k


def kernel(outputs, labels, cost_matrix):
    b, c = outputs.shape
    tc_rows = b - _SC_ROWS

    info = plsc.get_sparse_core_info()
    nw = info.num_cores * info.num_subcores
    cnt32, scm32 = _make_sc_kernel(
        nw, info.num_cores, b // nw, c, _SC_ROWS, tc_rows)(labels, outputs)

    acc = pl.pallas_call(
        _colmax_body,
        grid=(tc_rows // _BB,),
        in_specs=[pl.BlockSpec((_BB, c), lambda i: (i, 0))],
        out_specs=pl.BlockSpec((1, c), lambda i: (0, 0)),
        out_shape=jax.ShapeDtypeStruct((1, c), jnp.float32),
    )(outputs)

    loss = pl.pallas_call(
        _loss_body,
        in_specs=[
            pl.BlockSpec((c, c), lambda: (0, 0)),
            pl.BlockSpec(cnt32.shape, lambda: (0, 0)),
            pl.BlockSpec(scm32.shape, lambda: (0, 0)),
            pl.BlockSpec((1, c), lambda: (0, 0)),
        ],
        out_specs=pl.BlockSpec((1, 1), lambda: (0, 0)),
        out_shape=jax.ShapeDtypeStruct((1, 1), jnp.float32),
    )(cost_matrix, cnt32, scm32, acc)

    return loss[0, 0]


# speedup vs baseline: 1.2170x; 1.2170x over previous
import functools

import jax
import jax.numpy as jnp
from jax import lax
from jax.experimental import pallas as pl
from jax.experimental.pallas import tpu as pltpu
from jax.experimental.pallas import tpu_sc as plsc

_BB = 2048
_CP = 1008


def _main_body(x_ref, cm_ref, cnt_ref, o_ref, acc_ref):
    i = pl.program_id(0)
    x = x_ref[...]
    rowmax = jnp.max(x, axis=1, keepdims=True)
    cm = jnp.max(x - rowmax, axis=0, keepdims=True)

    @pl.when(i == 0)
    def _():
        acc_ref[...] = cm

    @pl.when(i != 0)
    def _():
        acc_ref[...] = jnp.maximum(acc_ref[...], cm)

    @pl.when(i == pl.num_programs(0) - 1)
    def _():
        c = cm_ref.shape[0]
        cnt = jnp.sum(cnt_ref[...], axis=0, keepdims=True)[:, :c]
        w = lax.dot_general(
            cnt,
            cm_ref[...],
            (((1,), (0,)), ((), ())),
            precision=lax.Precision.HIGHEST,
            preferred_element_type=jnp.float32,
        )
        mask = (acc_ref[...] == 0.0).astype(jnp.float32)
        o_ref[...] = -jnp.sum(w * mask, keepdims=True)


@functools.lru_cache(maxsize=None)
def _make_hist(nw, nc, nb):
    mesh = plsc.VectorSubcoreMesh(core_axis_name="c", subcore_axis_name="s")

    @functools.partial(
        pl.kernel,
        mesh=mesh,
        out_type=jax.ShapeDtypeStruct((nw, _CP), jnp.float32),
        scratch_types=[
            pltpu.VMEM((nb,), jnp.int32),
            pltpu.VMEM((_CP,), jnp.float32),
        ],
        compiler_params=pltpu.CompilerParams(needs_layout_passes=False),
    )
    def hist_k(labels_hbm, out_hbm, idx_v, hist_v):
        wid = lax.axis_index("s") * nc + lax.axis_index("c")
        pltpu.sync_copy(labels_hbm.at[pl.ds(wid * nb, nb)], idx_v)
        zeros = jnp.zeros((16,), jnp.float32)
        for j in range(_CP // 16):
            hist_v[pl.ds(j * 16, 16)] = zeros
        ones = jnp.ones((16,), jnp.float32)
        for i in range(nb // 16):
            plsc.addupdate_scatter(hist_v, [idx_v[pl.ds(i * 16, 16)]], ones)
        pltpu.sync_copy(hist_v, out_hbm.at[wid])

    return hist_k


def kernel(outputs, labels, cost_matrix):
    b, c = outputs.shape

    info = plsc.get_sparse_core_info()
    nw = info.num_cores * info.num_subcores
    cnt32 = _make_hist(nw, info.num_cores, b // nw)(labels)

    loss = pl.pallas_call(
        _main_body,
        grid=(b // _BB,),
        in_specs=[
            pl.BlockSpec((_BB, c), lambda i: (i, 0)),
            pl.BlockSpec((c, c), lambda i: (0, 0)),
            pl.BlockSpec(cnt32.shape, lambda i: (0, 0)),
        ],
        out_specs=pl.BlockSpec((1, 1), lambda i: (0, 0)),
        out_shape=jax.ShapeDtypeStruct((1, 1), jnp.float32),
        scratch_shapes=[pltpu.VMEM((1, c), jnp.float32)],
    )(outputs, cost_matrix, cnt32)

    return loss[0, 0]
